# baseline (speedup 1.0000x reference)
import functools

import numpy as np
import jax
import jax.numpy as jnp
from jax import lax
from jax.experimental import pallas as pl
from jax.experimental.pallas import tpu as pltpu
from jax.experimental.pallas import tpu_sc as plsc

N_HEADS = 8
N_LEVELS = 3
N_POINTS = 2
HEAD_DIM = 32
NCOMBO = N_HEADS * N_LEVELS * N_POINTS
NTAP = NCOMBO * 4
LVL_W = (64, 32, 16)
LVL_BASE = (0, 4096, 5120)
S_TOT = 5376

QT = 256
CHUNK = 8
NW = 32


def _prep_body(toff, q_ref, ref48_ref, wcat_ref, bcat_ref, idx_ref, wgt_ref):
    t = pl.program_id(0) + toff
    b_base = (t // (S_TOT // QT)) * S_TOT
    q = q_ref[...]
    g = lax.dot_general(q, wcat_ref[...], (((1,), (1,)), ((), ())),
                        preferred_element_type=jnp.float32) + bcat_ref[...]
    offx = jnp.tanh(g[:, :NCOMBO]) * 0.5
    offy = jnp.tanh(g[:, NCOMBO:2 * NCOMBO]) * 0.5
    ga = g[:, 2 * NCOMBO:]

    e = jnp.exp(ga)
    gi = lax.broadcasted_iota(jnp.int32, (NCOMBO, NCOMBO), 0) // 6
    gj = lax.broadcasted_iota(jnp.int32, (NCOMBO, NCOMBO), 1) // 6
    gm = (gi == gj).astype(jnp.float32)
    s = lax.dot_general(e, gm, (((1,), (0,)), ((), ())),
                        preferred_element_type=jnp.float32)
    attn48 = e / s

    ci = lax.broadcasted_iota(jnp.int32, (QT, NCOMBO), 1)
    h_i = ci // 6
    l_i = (ci % 6) // 2
    w_f = jnp.where(l_i == 0, float(LVL_W[0]),
                    jnp.where(l_i == 1, float(LVL_W[1]), float(LVL_W[2])))
    w_i = w_f.astype(jnp.int32)
    base_i = jnp.where(l_i == 0, LVL_BASE[0],
                       jnp.where(l_i == 1, LVL_BASE[1], LVL_BASE[2]))

    sx = ref48_ref[...] + offx
    sy = ref48_ref[...] + offy
    gx = sx * 2.0 - 1.0
    gy = sy * 2.0 - 1.0
    x = ((gx + 1.0) * w_f - 1.0) * 0.5
    y = ((gy + 1.0) * w_f - 1.0) * 0.5
    x0 = jnp.floor(x)
    y0 = jnp.floor(y)
    wx1 = x - x0
    wx0 = 1.0 - wx1
    wy1 = y - y0
    wy0 = 1.0 - wy1

    idx_parts, wgt_parts = [], []
    for dx, dy in ((0, 0), (1, 0), (0, 1), (1, 1)):
        ixf = x0 + float(dx)
        iyf = y0 + float(dy)
        valid = ((ixf >= 0.0) & (ixf <= w_f - 1.0)
                 & (iyf >= 0.0) & (iyf <= w_f - 1.0))
        ixi = jnp.clip(ixf, 0.0, w_f - 1.0).astype(jnp.int32)
        iyi = jnp.clip(iyf, 0.0, w_f - 1.0).astype(jnp.int32)
        row = ((b_base + base_i + iyi * w_i + ixi) * N_HEADS + h_i)
        cw = (wx1 if dx else wx0) * (wy1 if dy else wy0)
        wgt = attn48 * cw * valid.astype(jnp.float32)
        idx_parts.append(row)
        wgt_parts.append(wgt)
    idx_ref[...] = jnp.concatenate(idx_parts, axis=1)
    wgt_ref[...] = jnp.concatenate(wgt_parts, axis=1)


def _prep_call(qf, ref48, wcat, bcat, toff):
    bq = qf.shape[0]
    grid = (bq // QT,)
    return pl.pallas_call(
        functools.partial(_prep_body, toff),
        grid=grid,
        in_specs=[
            pl.BlockSpec((QT, 256), lambda i: (i, 0)),
            pl.BlockSpec((QT, NCOMBO), lambda i: (i, 0)),
            pl.BlockSpec((3 * NCOMBO, 256), lambda i: (0, 0)),
            pl.BlockSpec((1, 3 * NCOMBO), lambda i: (0, 0)),
        ],
        out_specs=[
            pl.BlockSpec((QT, NTAP), lambda i: (i, 0)),
            pl.BlockSpec((QT, NTAP), lambda i: (i, 0)),
        ],
        out_shape=[
            jax.ShapeDtypeStruct((bq, NTAP), jnp.int32),
            jax.ShapeDtypeStruct((bq, NTAP), jnp.float32),
        ],
    )(qf, ref48, wcat, bcat)


VT = 384


def _vproj_body(v_ref, w_ref, b_ref, o_ref):
    f = v_ref[0].astype(jnp.bfloat16)
    o = lax.dot_general(f, w_ref[...], (((1,), (0,)), ((), ())),
                        preferred_element_type=jnp.float32)
    o_ref[0] = (o + b_ref[...]).astype(jnp.bfloat16)


def _vproj_call(vt, w_val_t, b_val):
    b = vt.shape[0]
    grid = (b, S_TOT // VT)
    return pl.pallas_call(
        _vproj_body,
        grid=grid,
        in_specs=[
            pl.BlockSpec((1, VT, 256), lambda i, j: (i, j, 0)),
            pl.BlockSpec((256, 256), lambda i, j: (0, 0)),
            pl.BlockSpec((1, 256), lambda i, j: (0, 0)),
        ],
        out_specs=pl.BlockSpec((1, VT, 256), lambda i, j: (i, j, 0)),
        out_shape=jax.ShapeDtypeStruct((b, S_TOT, 256), jnp.bfloat16),
    )(vt, w_val_t, b_val)


def _sc_body(table, idxh, wh, outh, idx_v, rows_v, w_v, out_v, sem0, sem1):
    wid = lax.axis_index("s") * 2 + lax.axis_index("c")
    n_chunks = idxh.shape[0] // NW
    base = wid * n_chunks
    sems = (sem0, sem1)

    def gathers(buf):
        for j in range(12):
            yield (table.at[idx_v.at[buf, j]],
                   rows_v.at[buf, pl.ds(j * 128, 128)])

    def fire(ci, buf):
        ck = base + ci
        pltpu.sync_copy(idxh.at[ck], idx_v.at[buf])
        pltpu.sync_copy(wh.at[pl.ds(ck * CHUNK, CHUNK)], w_v.at[buf])
        for src, dst in gathers(buf):
            pltpu.async_copy(src, dst, sems[buf])

    def consume(ci, buf):
        ck = base + ci
        for src, dst in gathers(buf):
            pltpu.make_async_copy(src, dst, sems[buf]).wait()

        def q_body(q, c2):
            ws = [w_v[buf, q, pl.ds(k * 16, 16)] for k in range(NTAP // 16)]
            for h in range(N_HEADS):
                acc0 = jnp.zeros((16,), jnp.float32)
                acc1 = jnp.zeros((16,), jnp.float32)
                for corner in range(4):
                    for j in range(6):
                        col = corner * NCOMBO + h * 6 + j
                        wv = jnp.full((16,), ws[col // 16][col % 16])
                        r = q * NTAP + col
                        wd = plsc.bitcast(rows_v[buf, r, :], jnp.int32)
                        f_ev = plsc.bitcast(wd << 16, jnp.float32)
                        f_od = plsc.bitcast(wd & jnp.int32(-65536), jnp.float32)
                        acc0 = acc0 + f_ev * wv
                        acc1 = acc1 + f_od * wv
                out_v[q, pl.ds(h * 32, 16)] = acc0
                out_v[q, pl.ds(h * 32 + 16, 16)] = acc1
            return c2

        lax.fori_loop(0, CHUNK, q_body, 0)
        pltpu.sync_copy(out_v, outh.at[pl.ds(ck * CHUNK, CHUNK)])

    fire(0, 0)

    def it_body(it, carry):
        i0 = it * 2
        fire(i0 + 1, 1)
        consume(i0, 0)

        @pl.when(i0 + 2 < n_chunks)
        def _():
            fire(i0 + 2, 0)

        consume(i0 + 1, 1)
        return carry

    lax.fori_loop(0, n_chunks // 2, it_body, 0)


def _sc_call(table, idx3, w3, bq):
    mesh = plsc.VectorSubcoreMesh(core_axis_name="c", subcore_axis_name="s")
    f = pl.kernel(
        _sc_body,
        out_type=jax.ShapeDtypeStruct((bq, 256), jnp.float32),
        mesh=mesh,
        scratch_types=[
            pltpu.VMEM((2, 12, 128), jnp.int32),
            pltpu.VMEM((2, CHUNK * NTAP, HEAD_DIM), jnp.bfloat16),
            pltpu.VMEM((2, CHUNK, NTAP), jnp.float32),
            pltpu.VMEM((CHUNK, 256), jnp.float32),
            pltpu.SemaphoreType.DMA,
            pltpu.SemaphoreType.DMA,
        ],
        compiler_params=pltpu.CompilerParams(use_tc_tiling_on_sc=False,
                                             needs_layout_passes=False),
    )
    return f(table, idx3, w3)


def _oproj_body(a_ref, w_ref, b_ref, o_ref):
    o = lax.dot_general(a_ref[...], w_ref[...], (((1,), (1,)), ((), ())),
                        preferred_element_type=jnp.float32)
    o_ref[...] = o + b_ref[...]


def _oproj_call(acc, w_out, b_out):
    bq = acc.shape[0]
    return pl.pallas_call(
        _oproj_body,
        grid=(bq // QT,),
        in_specs=[
            pl.BlockSpec((QT, 256), lambda i: (i, 0)),
            pl.BlockSpec((256, 256), lambda i: (0, 0)),
            pl.BlockSpec((1, 256), lambda i: (0, 0)),
        ],
        out_specs=pl.BlockSpec((QT, 256), lambda i: (i, 0)),
        out_shape=jax.ShapeDtypeStruct((bq, 256), jnp.float32),
    )(acc, w_out, b_out)


def kernel(query, value_l0, value_l1, value_l2, reference_points,
           W_off, b_off, W_attn, b_attn, W_val, b_val, W_out, b_out):
    b, q, d = query.shape
    bq = b * q
    hq = bq // 2
    qf = query.reshape(bq, d)
    ref48 = jnp.broadcast_to(reference_points.reshape(bq, 1, 2),
                             (bq, NCOMBO // 2, 2)).reshape(bq, NCOMBO)
    wcat = jnp.concatenate([W_off[0::2], W_off[1::2], W_attn], axis=0)
    bcat = jnp.concatenate([b_off[0::2], b_off[1::2], b_attn],
                           axis=0).reshape(1, 3 * NCOMBO)

    vcat = jnp.concatenate([
        value_l0.reshape(b, d, LVL_W[0] * LVL_W[0]),
        value_l1.reshape(b, d, LVL_W[1] * LVL_W[1]),
        value_l2.reshape(b, d, LVL_W[2] * LVL_W[2]),
    ], axis=2).transpose(0, 2, 1)
    table = _vproj_call(vcat, W_val.T.astype(jnp.bfloat16), b_val.reshape(1, d))
    table = table.reshape(b * S_TOT * N_HEADS, HEAD_DIM)

    j = np.arange(HEAD_DIM)
    tr = 2 * (j % 16) + (j // 16)
    perm = (np.arange(N_HEADS)[:, None] * HEAD_DIM + tr[None, :]).reshape(-1)
    w_out_p = W_out[:, jnp.asarray(perm)]
    b_out2 = b_out.reshape(1, d)

    idx0, wgt0 = _prep_call(qf[:hq], ref48[:hq], wcat, bcat, 0)
    acc0 = _sc_call(table, idx0.reshape(hq // CHUNK, 12, 128), wgt0, hq)
    idx1, wgt1 = _prep_call(qf[hq:], ref48[hq:], wcat, bcat, hq // QT)
    acc1 = _sc_call(table, idx1.reshape(hq // CHUNK, 12, 128), wgt1, hq)
    out0 = _oproj_call(acc0, w_out_p, b_out2)
    out1 = _oproj_call(acc1, w_out_p, b_out2)
    return jnp.concatenate([out0, out1], axis=0).reshape(b, q, d)


# speedup vs baseline: 1.0875x; 1.0875x over previous
import functools

import numpy as np
import jax
import jax.numpy as jnp
from jax import lax
from jax.experimental import pallas as pl
from jax.experimental.pallas import tpu as pltpu
from jax.experimental.pallas import tpu_sc as plsc

N_HEADS = 8
N_LEVELS = 3
N_POINTS = 2
HEAD_DIM = 32
NCOMBO = N_HEADS * N_LEVELS * N_POINTS
NTAP = NCOMBO * 4
LVL_W = (64, 32, 16)
LVL_BASE = (0, 4096, 5120)
S_TOT = 5376

QT = 256
CHUNK = 8
NW = 32


def _prep_body(toff, q_ref, ref48_ref, wcat_ref, bcat_ref, idx_ref, wgt_ref):
    t = pl.program_id(0) + toff
    b_base = (t // (S_TOT // QT)) * S_TOT
    q = q_ref[...]
    g = lax.dot_general(q, wcat_ref[...], (((1,), (1,)), ((), ())),
                        preferred_element_type=jnp.float32) + bcat_ref[...]
    offx = jnp.tanh(g[:, :NCOMBO]) * 0.5
    offy = jnp.tanh(g[:, NCOMBO:2 * NCOMBO]) * 0.5
    ga = g[:, 2 * NCOMBO:]

    e = jnp.exp(ga)
    gi = lax.broadcasted_iota(jnp.int32, (NCOMBO, NCOMBO), 0) // 6
    gj = lax.broadcasted_iota(jnp.int32, (NCOMBO, NCOMBO), 1) // 6
    gm = (gi == gj).astype(jnp.float32)
    s = lax.dot_general(e, gm, (((1,), (0,)), ((), ())),
                        preferred_element_type=jnp.float32)
    attn48 = e / s

    ci = lax.broadcasted_iota(jnp.int32, (QT, NCOMBO), 1)
    h_i = ci // 6
    l_i = (ci % 6) // 2
    w_f = jnp.where(l_i == 0, float(LVL_W[0]),
                    jnp.where(l_i == 1, float(LVL_W[1]), float(LVL_W[2])))
    w_i = w_f.astype(jnp.int32)
    base_i = jnp.where(l_i == 0, LVL_BASE[0],
                       jnp.where(l_i == 1, LVL_BASE[1], LVL_BASE[2]))

    sx = ref48_ref[...] + offx
    sy = ref48_ref[...] + offy
    gx = sx * 2.0 - 1.0
    gy = sy * 2.0 - 1.0
    x = ((gx + 1.0) * w_f - 1.0) * 0.5
    y = ((gy + 1.0) * w_f - 1.0) * 0.5
    x0 = jnp.floor(x)
    y0 = jnp.floor(y)
    wx1 = x - x0
    wx0 = 1.0 - wx1
    wy1 = y - y0
    wy0 = 1.0 - wy1

    idx_parts, wgt_parts = [], []
    for dx, dy in ((0, 0), (1, 0), (0, 1), (1, 1)):
        ixf = x0 + float(dx)
        iyf = y0 + float(dy)
        valid = ((ixf >= 0.0) & (ixf <= w_f - 1.0)
                 & (iyf >= 0.0) & (iyf <= w_f - 1.0))
        ixi = jnp.clip(ixf, 0.0, w_f - 1.0).astype(jnp.int32)
        iyi = jnp.clip(iyf, 0.0, w_f - 1.0).astype(jnp.int32)
        row = ((b_base + base_i + iyi * w_i + ixi) * N_HEADS + h_i)
        cw = (wx1 if dx else wx0) * (wy1 if dy else wy0)
        wgt = attn48 * cw * valid.astype(jnp.float32)
        idx_parts.append(row)
        wgt_parts.append(wgt)
    idx_ref[...] = jnp.concatenate(idx_parts, axis=1)
    wgt_ref[...] = jnp.concatenate(wgt_parts, axis=1)


def _prep_call(qf, ref48, wcat, bcat, toff):
    bq = qf.shape[0]
    grid = (bq // QT,)
    return pl.pallas_call(
        functools.partial(_prep_body, toff),
        grid=grid,
        in_specs=[
            pl.BlockSpec((QT, 256), lambda i: (i, 0)),
            pl.BlockSpec((QT, NCOMBO), lambda i: (i, 0)),
            pl.BlockSpec((3 * NCOMBO, 256), lambda i: (0, 0)),
            pl.BlockSpec((1, 3 * NCOMBO), lambda i: (0, 0)),
        ],
        out_specs=[
            pl.BlockSpec((QT, NTAP), lambda i: (i, 0)),
            pl.BlockSpec((QT, NTAP), lambda i: (i, 0)),
        ],
        out_shape=[
            jax.ShapeDtypeStruct((bq, NTAP), jnp.int32),
            jax.ShapeDtypeStruct((bq, NTAP), jnp.float32),
        ],
    )(qf, ref48, wcat, bcat)


VT = 384


def _vproj_body(v_ref, w_ref, b_ref, o_ref):
    f = v_ref[0].astype(jnp.bfloat16)
    o = lax.dot_general(f, w_ref[...], (((1,), (0,)), ((), ())),
                        preferred_element_type=jnp.float32)
    o_ref[0] = (o + b_ref[...]).astype(jnp.bfloat16)


def _vproj_call(vt, w_val_t, b_val):
    b = vt.shape[0]
    grid = (b, S_TOT // VT)
    return pl.pallas_call(
        _vproj_body,
        grid=grid,
        in_specs=[
            pl.BlockSpec((1, VT, 256), lambda i, j: (i, j, 0)),
            pl.BlockSpec((256, 256), lambda i, j: (0, 0)),
            pl.BlockSpec((1, 256), lambda i, j: (0, 0)),
        ],
        out_specs=pl.BlockSpec((1, VT, 256), lambda i, j: (i, j, 0)),
        out_shape=jax.ShapeDtypeStruct((b, S_TOT, 256), jnp.bfloat16),
    )(vt, w_val_t, b_val)


def _sc_body(table, idxh, wh, outh, idx_v, rows_v, w_v, out_v,
             sem0, sem1, semw0, semw1, semo0, semo1):
    wid = lax.axis_index("s") * 2 + lax.axis_index("c")
    n_chunks = idxh.shape[0] // NW
    base = wid * n_chunks
    sems = (sem0, sem1)
    semw = (semw0, semw1)
    semo = (semo0, semo1)

    def gathers(buf):
        for j in range(12):
            yield (table.at[idx_v.at[buf, j]],
                   rows_v.at[buf, pl.ds(j * 128, 128)])

    def wcopy(ci, buf):
        return pltpu.make_async_copy(
            wh.at[pl.ds((base + ci) * CHUNK, CHUNK)], w_v.at[buf], semw[buf])

    def ocopy(ci, buf):
        return pltpu.make_async_copy(
            out_v.at[buf], outh.at[pl.ds((base + ci) * CHUNK, CHUNK)],
            semo[buf])

    def fire(ci, buf):
        pltpu.sync_copy(idxh.at[base + ci], idx_v.at[buf])
        wcopy(ci, buf).start()
        for src, dst in gathers(buf):
            pltpu.async_copy(src, dst, sems[buf])

    def consume(ci, buf):
        for src, dst in gathers(buf):
            pltpu.make_async_copy(src, dst, sems[buf]).wait()
        wcopy(ci, buf).wait()

        @pl.when(ci >= 2)
        def _():
            ocopy(ci - 2, buf).wait()

        def q_body(q, c2):
            ws = [w_v[buf, q, pl.ds(k * 16, 16)] for k in range(NTAP // 16)]
            for h in range(N_HEADS):
                acc0 = jnp.zeros((16,), jnp.float32)
                acc1 = jnp.zeros((16,), jnp.float32)
                for corner in range(4):
                    for j in range(6):
                        col = corner * NCOMBO + h * 6 + j
                        wv = jnp.full((16,), ws[col // 16][col % 16])
                        r = q * NTAP + col
                        wd = plsc.bitcast(rows_v[buf, r, :], jnp.int32)
                        f_ev = plsc.bitcast(wd << 16, jnp.float32)
                        f_od = plsc.bitcast(wd & jnp.int32(-65536), jnp.float32)
                        acc0 = acc0 + f_ev * wv
                        acc1 = acc1 + f_od * wv
                out_v[buf, q, pl.ds(h * 32, 16)] = acc0
                out_v[buf, q, pl.ds(h * 32 + 16, 16)] = acc1
            return c2

        lax.fori_loop(0, CHUNK, q_body, 0)
        ocopy(ci, buf).start()

    fire(0, 0)

    def it_body(it, carry):
        i0 = it * 2
        fire(i0 + 1, 1)
        consume(i0, 0)

        @pl.when(i0 + 2 < n_chunks)
        def _():
            fire(i0 + 2, 0)

        consume(i0 + 1, 1)
        return carry

    lax.fori_loop(0, n_chunks // 2, it_body, 0)
    ocopy(n_chunks - 2, 0).wait()
    ocopy(n_chunks - 1, 1).wait()


def _sc_call(table, idx3, w3, bq):
    mesh = plsc.VectorSubcoreMesh(core_axis_name="c", subcore_axis_name="s")
    f = pl.kernel(
        _sc_body,
        out_type=jax.ShapeDtypeStruct((bq, 256), jnp.float32),
        mesh=mesh,
        scratch_types=[
            pltpu.VMEM((2, 12, 128), jnp.int32),
            pltpu.VMEM((2, CHUNK * NTAP, HEAD_DIM), jnp.bfloat16),
            pltpu.VMEM((2, CHUNK, NTAP), jnp.float32),
            pltpu.VMEM((2, CHUNK, 256), jnp.float32),
            pltpu.SemaphoreType.DMA,
            pltpu.SemaphoreType.DMA,
            pltpu.SemaphoreType.DMA,
            pltpu.SemaphoreType.DMA,
            pltpu.SemaphoreType.DMA,
            pltpu.SemaphoreType.DMA,
        ],
        compiler_params=pltpu.CompilerParams(use_tc_tiling_on_sc=False,
                                             needs_layout_passes=False),
    )
    return f(table, idx3, w3)


def _oproj_body(a_ref, w_ref, b_ref, o_ref):
    o = lax.dot_general(a_ref[...], w_ref[...], (((1,), (1,)), ((), ())),
                        preferred_element_type=jnp.float32)
    o_ref[...] = o + b_ref[...]


def _oproj_call(acc, w_out, b_out):
    bq = acc.shape[0]
    return pl.pallas_call(
        _oproj_body,
        grid=(bq // QT,),
        in_specs=[
            pl.BlockSpec((QT, 256), lambda i: (i, 0)),
            pl.BlockSpec((256, 256), lambda i: (0, 0)),
            pl.BlockSpec((1, 256), lambda i: (0, 0)),
        ],
        out_specs=pl.BlockSpec((QT, 256), lambda i: (i, 0)),
        out_shape=jax.ShapeDtypeStruct((bq, 256), jnp.float32),
    )(acc, w_out, b_out)


def kernel(query, value_l0, value_l1, value_l2, reference_points,
           W_off, b_off, W_attn, b_attn, W_val, b_val, W_out, b_out):
    b, q, d = query.shape
    bq = b * q
    hq = bq // 2
    qf = query.reshape(bq, d)
    ref48 = jnp.broadcast_to(reference_points.reshape(bq, 1, 2),
                             (bq, NCOMBO // 2, 2)).reshape(bq, NCOMBO)
    wcat = jnp.concatenate([W_off[0::2], W_off[1::2], W_attn], axis=0)
    bcat = jnp.concatenate([b_off[0::2], b_off[1::2], b_attn],
                           axis=0).reshape(1, 3 * NCOMBO)

    vcat = jnp.concatenate([
        value_l0.reshape(b, d, LVL_W[0] * LVL_W[0]),
        value_l1.reshape(b, d, LVL_W[1] * LVL_W[1]),
        value_l2.reshape(b, d, LVL_W[2] * LVL_W[2]),
    ], axis=2).transpose(0, 2, 1)
    table = _vproj_call(vcat, W_val.T.astype(jnp.bfloat16), b_val.reshape(1, d))
    table = table.reshape(b * S_TOT * N_HEADS, HEAD_DIM)

    j = np.arange(HEAD_DIM)
    tr = 2 * (j % 16) + (j // 16)
    perm = (np.arange(N_HEADS)[:, None] * HEAD_DIM + tr[None, :]).reshape(-1)
    w_out_p = W_out[:, jnp.asarray(perm)]
    b_out2 = b_out.reshape(1, d)

    idx, wgt = _prep_call(qf, ref48, wcat, bcat, 0)
    acc = _sc_call(table, idx.reshape(bq // CHUNK, 12, 128), wgt, bq)
    out = _oproj_call(acc, w_out_p, b_out2)
    return out.reshape(b, q, d)


# speedup vs baseline: 1.0886x; 1.0011x over previous
import functools

import numpy as np
import jax
import jax.numpy as jnp
from jax import lax
from jax.experimental import pallas as pl
from jax.experimental.pallas import tpu as pltpu
from jax.experimental.pallas import tpu_sc as plsc

N_HEADS = 8
N_LEVELS = 3
N_POINTS = 2
HEAD_DIM = 32
NCOMBO = N_HEADS * N_LEVELS * N_POINTS
NTAP = NCOMBO * 4
LVL_W = (64, 32, 16)
LVL_BASE = (0, 4096, 5120)
S_TOT = 5376

QT = 256
CHUNK = 8
NW = 32


def _prep_body(toff, q_ref, ref48_ref, wcat_ref, bcat_ref, idx_ref, wgt_ref):
    t = pl.program_id(0) + toff
    b_base = (t // (S_TOT // QT)) * S_TOT
    q = q_ref[...]
    g = lax.dot_general(q, wcat_ref[...], (((1,), (1,)), ((), ())),
                        preferred_element_type=jnp.float32) + bcat_ref[...]
    offx = jnp.tanh(g[:, :NCOMBO]) * 0.5
    offy = jnp.tanh(g[:, NCOMBO:2 * NCOMBO]) * 0.5
    ga = g[:, 2 * NCOMBO:]

    e = jnp.exp(ga)
    gi = lax.broadcasted_iota(jnp.int32, (NCOMBO, NCOMBO), 0) // 6
    gj = lax.broadcasted_iota(jnp.int32, (NCOMBO, NCOMBO), 1) // 6
    gm = (gi == gj).astype(jnp.float32)
    s = lax.dot_general(e, gm, (((1,), (0,)), ((), ())),
                        preferred_element_type=jnp.float32)
    attn48 = e / s

    ci = lax.broadcasted_iota(jnp.int32, (QT, NCOMBO), 1)
    h_i = ci // 6
    l_i = (ci % 6) // 2
    w_f = jnp.where(l_i == 0, float(LVL_W[0]),
                    jnp.where(l_i == 1, float(LVL_W[1]), float(LVL_W[2])))
    w_i = w_f.astype(jnp.int32)
    base_i = jnp.where(l_i == 0, LVL_BASE[0],
                       jnp.where(l_i == 1, LVL_BASE[1], LVL_BASE[2]))

    sx = ref48_ref[...] + offx
    sy = ref48_ref[...] + offy
    gx = sx * 2.0 - 1.0
    gy = sy * 2.0 - 1.0
    x = ((gx + 1.0) * w_f - 1.0) * 0.5
    y = ((gy + 1.0) * w_f - 1.0) * 0.5
    x0 = jnp.floor(x)
    y0 = jnp.floor(y)
    wx1 = x - x0
    wx0 = 1.0 - wx1
    wy1 = y - y0
    wy0 = 1.0 - wy1

    idx_parts, wgt_parts = [], []
    for dx, dy in ((0, 0), (1, 0), (0, 1), (1, 1)):
        ixf = x0 + float(dx)
        iyf = y0 + float(dy)
        valid = ((ixf >= 0.0) & (ixf <= w_f - 1.0)
                 & (iyf >= 0.0) & (iyf <= w_f - 1.0))
        ixi = jnp.clip(ixf, 0.0, w_f - 1.0).astype(jnp.int32)
        iyi = jnp.clip(iyf, 0.0, w_f - 1.0).astype(jnp.int32)
        row = ((b_base + base_i + iyi * w_i + ixi) * N_HEADS + h_i)
        cw = (wx1 if dx else wx0) * (wy1 if dy else wy0)
        wgt = attn48 * cw * valid.astype(jnp.float32)
        idx_parts.append(row)
        wgt_parts.append(wgt)
    idx_ref[...] = jnp.concatenate(idx_parts, axis=1)
    wgt_ref[...] = jnp.concatenate(wgt_parts, axis=1)


def _prep_call(qf, ref48, wcat, bcat, toff):
    bq = qf.shape[0]
    grid = (bq // QT,)
    return pl.pallas_call(
        functools.partial(_prep_body, toff),
        grid=grid,
        in_specs=[
            pl.BlockSpec((QT, 256), lambda i: (i, 0)),
            pl.BlockSpec((QT, NCOMBO), lambda i: (i, 0)),
            pl.BlockSpec((3 * NCOMBO, 256), lambda i: (0, 0)),
            pl.BlockSpec((1, 3 * NCOMBO), lambda i: (0, 0)),
        ],
        out_specs=[
            pl.BlockSpec((QT, NTAP), lambda i: (i, 0)),
            pl.BlockSpec((QT, NTAP), lambda i: (i, 0)),
        ],
        out_shape=[
            jax.ShapeDtypeStruct((bq, NTAP), jnp.int32),
            jax.ShapeDtypeStruct((bq, NTAP), jnp.float32),
        ],
    )(qf, ref48, wcat, bcat)


VT = 384


def _vproj_body(v_ref, w_ref, b_ref, o_ref):
    f = v_ref[0].astype(jnp.bfloat16)
    o = lax.dot_general(f, w_ref[...], (((1,), (0,)), ((), ())),
                        preferred_element_type=jnp.float32)
    o_ref[0] = (o + b_ref[...]).astype(jnp.bfloat16)


def _vproj_call(vt, w_val_t, b_val):
    b = vt.shape[0]
    grid = (b, S_TOT // VT)
    return pl.pallas_call(
        _vproj_body,
        grid=grid,
        in_specs=[
            pl.BlockSpec((1, VT, 256), lambda i, j: (i, j, 0)),
            pl.BlockSpec((256, 256), lambda i, j: (0, 0)),
            pl.BlockSpec((1, 256), lambda i, j: (0, 0)),
        ],
        out_specs=pl.BlockSpec((1, VT, 256), lambda i, j: (i, j, 0)),
        out_shape=jax.ShapeDtypeStruct((b, S_TOT, 256), jnp.bfloat16),
    )(vt, w_val_t, b_val)


def _sc_body(table, idxh, wh, outh, idx_v, rows_v, w_v, out_v,
             sem0, sem1, semw0, semw1, semo0, semo1, semi0, semi1):
    wid = lax.axis_index("s") * 2 + lax.axis_index("c")
    n_chunks = idxh.shape[0] // NW
    base = wid * n_chunks
    sems = (sem0, sem1)
    semw = (semw0, semw1)
    semo = (semo0, semo1)
    semi = (semi0, semi1)

    def gathers(buf):
        for j in range(12):
            yield (table.at[idx_v.at[buf, j]],
                   rows_v.at[buf, pl.ds(j * 128, 128)])

    def icopy(ci, buf):
        return pltpu.make_async_copy(idxh.at[base + ci], idx_v.at[buf],
                                     semi[buf])

    def wcopy(ci, buf):
        return pltpu.make_async_copy(
            wh.at[pl.ds((base + ci) * CHUNK, CHUNK)], w_v.at[buf], semw[buf])

    def ocopy(ci, buf):
        return pltpu.make_async_copy(
            out_v.at[buf], outh.at[pl.ds((base + ci) * CHUNK, CHUNK)],
            semo[buf])

    def fire(ci, buf):
        icopy(ci, buf).wait()
        wcopy(ci, buf).start()
        for src, dst in gathers(buf):
            pltpu.async_copy(src, dst, sems[buf])

    def consume(ci, buf):
        for src, dst in gathers(buf):
            pltpu.make_async_copy(src, dst, sems[buf]).wait()

        @pl.when(ci + 2 < n_chunks)
        def _():
            icopy(ci + 2, buf).start()

        wcopy(ci, buf).wait()

        @pl.when(ci >= 2)
        def _():
            ocopy(ci - 2, buf).wait()

        def q_body(q, c2):
            ws = [w_v[buf, q, pl.ds(k * 16, 16)] for k in range(NTAP // 16)]
            for h in range(N_HEADS):
                acc0 = jnp.zeros((16,), jnp.float32)
                acc1 = jnp.zeros((16,), jnp.float32)
                for corner in range(4):
                    for j in range(6):
                        col = corner * NCOMBO + h * 6 + j
                        wv = jnp.full((16,), ws[col // 16][col % 16])
                        r = q * NTAP + col
                        wd = plsc.bitcast(rows_v[buf, r, :], jnp.int32)
                        f_ev = plsc.bitcast(wd << 16, jnp.float32)
                        f_od = plsc.bitcast(wd & jnp.int32(-65536), jnp.float32)
                        acc0 = acc0 + f_ev * wv
                        acc1 = acc1 + f_od * wv
                out_v[buf, q, pl.ds(h * 32, 16)] = acc0
                out_v[buf, q, pl.ds(h * 32 + 16, 16)] = acc1
            return c2

        lax.fori_loop(0, CHUNK, q_body, 0)
        ocopy(ci, buf).start()

    icopy(0, 0).start()
    icopy(1, 1).start()
    fire(0, 0)

    def it_body(it, carry):
        i0 = it * 2
        fire(i0 + 1, 1)
        consume(i0, 0)

        @pl.when(i0 + 2 < n_chunks)
        def _():
            fire(i0 + 2, 0)

        consume(i0 + 1, 1)
        return carry

    lax.fori_loop(0, n_chunks // 2, it_body, 0)
    ocopy(n_chunks - 2, 0).wait()
    ocopy(n_chunks - 1, 1).wait()


def _sc_call(table, idx3, w3, bq):
    mesh = plsc.VectorSubcoreMesh(core_axis_name="c", subcore_axis_name="s")
    f = pl.kernel(
        _sc_body,
        out_type=jax.ShapeDtypeStruct((bq, 256), jnp.float32),
        mesh=mesh,
        scratch_types=[
            pltpu.VMEM((2, 12, 128), jnp.int32),
            pltpu.VMEM((2, CHUNK * NTAP, HEAD_DIM), jnp.bfloat16),
            pltpu.VMEM((2, CHUNK, NTAP), jnp.float32),
            pltpu.VMEM((2, CHUNK, 256), jnp.float32),
            pltpu.SemaphoreType.DMA,
            pltpu.SemaphoreType.DMA,
            pltpu.SemaphoreType.DMA,
            pltpu.SemaphoreType.DMA,
            pltpu.SemaphoreType.DMA,
            pltpu.SemaphoreType.DMA,
            pltpu.SemaphoreType.DMA,
            pltpu.SemaphoreType.DMA,
        ],
        compiler_params=pltpu.CompilerParams(use_tc_tiling_on_sc=False,
                                             needs_layout_passes=False),
    )
    return f(table, idx3, w3)


def _oproj_body(a_ref, w_ref, b_ref, o_ref):
    o = lax.dot_general(a_ref[...], w_ref[...], (((1,), (1,)), ((), ())),
                        preferred_element_type=jnp.float32)
    o_ref[...] = o + b_ref[...]


def _oproj_call(acc, w_out, b_out):
    bq = acc.shape[0]
    return pl.pallas_call(
        _oproj_body,
        grid=(bq // QT,),
        in_specs=[
            pl.BlockSpec((QT, 256), lambda i: (i, 0)),
            pl.BlockSpec((256, 256), lambda i: (0, 0)),
            pl.BlockSpec((1, 256), lambda i: (0, 0)),
        ],
        out_specs=pl.BlockSpec((QT, 256), lambda i: (i, 0)),
        out_shape=jax.ShapeDtypeStruct((bq, 256), jnp.float32),
    )(acc, w_out, b_out)


def kernel(query, value_l0, value_l1, value_l2, reference_points,
           W_off, b_off, W_attn, b_attn, W_val, b_val, W_out, b_out):
    b, q, d = query.shape
    bq = b * q
    hq = bq // 2
    qf = query.reshape(bq, d)
    ref48 = jnp.broadcast_to(reference_points.reshape(bq, 1, 2),
                             (bq, NCOMBO // 2, 2)).reshape(bq, NCOMBO)
    wcat = jnp.concatenate([W_off[0::2], W_off[1::2], W_attn], axis=0)
    bcat = jnp.concatenate([b_off[0::2], b_off[1::2], b_attn],
                           axis=0).reshape(1, 3 * NCOMBO)

    vcat = jnp.concatenate([
        value_l0.reshape(b, d, LVL_W[0] * LVL_W[0]),
        value_l1.reshape(b, d, LVL_W[1] * LVL_W[1]),
        value_l2.reshape(b, d, LVL_W[2] * LVL_W[2]),
    ], axis=2).transpose(0, 2, 1)
    table = _vproj_call(vcat, W_val.T.astype(jnp.bfloat16), b_val.reshape(1, d))
    table = table.reshape(b * S_TOT * N_HEADS, HEAD_DIM)

    j = np.arange(HEAD_DIM)
    tr = 2 * (j % 16) + (j // 16)
    perm = (np.arange(N_HEADS)[:, None] * HEAD_DIM + tr[None, :]).reshape(-1)
    w_out_p = W_out[:, jnp.asarray(perm)]
    b_out2 = b_out.reshape(1, d)

    idx, wgt = _prep_call(qf, ref48, wcat, bcat, 0)
    acc = _sc_call(table, idx.reshape(bq // CHUNK, 12, 128), wgt, bq)
    out = _oproj_call(acc, w_out_p, b_out2)
    return out.reshape(b, q, d)


# speedup vs baseline: 1.0911x; 1.0023x over previous
import functools

import numpy as np
import jax
import jax.numpy as jnp
from jax import lax
from jax.experimental import pallas as pl
from jax.experimental.pallas import tpu as pltpu
from jax.experimental.pallas import tpu_sc as plsc

N_HEADS = 8
N_LEVELS = 3
N_POINTS = 2
HEAD_DIM = 32
NCOMBO = N_HEADS * N_LEVELS * N_POINTS
NTAP = NCOMBO * 4
LVL_W = (64, 32, 16)
LVL_BASE = (0, 4096, 5120)
S_TOT = 5376

QT = 256
CHUNK = 8
NW = 32


def _prep_body(toff, q_ref, ref48_ref, wcat_ref, bcat_ref, lc_ref, gm_ref,
               idx_ref, wgt_ref):
    t = pl.program_id(0) + toff
    b_base = ((t // (S_TOT // QT)) * S_TOT).astype(jnp.float32)
    q = q_ref[...]
    g = lax.dot_general(q, wcat_ref[...], (((1,), (1,)), ((), ())),
                        preferred_element_type=jnp.float32) + bcat_ref[...]
    offx = jnp.tanh(g[:, :NCOMBO]) * 0.5
    offy = jnp.tanh(g[:, NCOMBO:2 * NCOMBO]) * 0.5
    ga = g[:, 2 * NCOMBO:]

    e = jnp.exp(ga)
    s = lax.dot_general(e, gm_ref[...], (((1,), (0,)), ((), ())),
                        preferred_element_type=jnp.float32)
    attn48 = e / s

    w_f = lc_ref[0:1, :]
    base_f = lc_ref[1:2, :]
    h_f = lc_ref[2:3, :]

    sx = ref48_ref[...] + offx
    sy = ref48_ref[...] + offy
    gx = sx * 2.0 - 1.0
    gy = sy * 2.0 - 1.0
    x = ((gx + 1.0) * w_f - 1.0) * 0.5
    y = ((gy + 1.0) * w_f - 1.0) * 0.5
    x0 = jnp.floor(x)
    y0 = jnp.floor(y)
    wx1 = x - x0
    wx0 = 1.0 - wx1
    wy1 = y - y0
    wy0 = 1.0 - wy1
    rbase = (b_base + base_f) * float(N_HEADS) + h_f

    idx_parts, wgt_parts = [], []
    for dx, dy in ((0, 0), (1, 0), (0, 1), (1, 1)):
        ixf = x0 + float(dx)
        iyf = y0 + float(dy)
        valid = ((ixf >= 0.0) & (ixf <= w_f - 1.0)
                 & (iyf >= 0.0) & (iyf <= w_f - 1.0))
        ixc = jnp.clip(ixf, 0.0, w_f - 1.0)
        iyc = jnp.clip(iyf, 0.0, w_f - 1.0)
        row = rbase + (iyc * w_f + ixc) * float(N_HEADS)
        cw = (wx1 if dx else wx0) * (wy1 if dy else wy0)
        wgt = attn48 * cw * valid.astype(jnp.float32)
        idx_parts.append(row.astype(jnp.int32))
        wgt_parts.append(wgt)
    idx_ref[...] = jnp.concatenate(idx_parts, axis=1)
    wgt_ref[...] = jnp.concatenate(wgt_parts, axis=1)


def _lane_consts():
    c = np.arange(NCOMBO)
    l = (c % 6) // 2
    w = np.array(LVL_W)[l]
    base = np.array(LVL_BASE)[l]
    h = c // 6
    return jnp.asarray(np.stack([w, base, h]).astype(np.float32))


def _group_mask():
    gi = np.arange(NCOMBO)[:, None] // 6
    gj = np.arange(NCOMBO)[None, :] // 6
    return jnp.asarray((gi == gj).astype(np.float32))


def _prep_call(qf, ref48, wcat, bcat, toff):
    bq = qf.shape[0]
    grid = (bq // QT,)
    return pl.pallas_call(
        functools.partial(_prep_body, toff),
        grid=grid,
        in_specs=[
            pl.BlockSpec((QT, 256), lambda i: (i, 0)),
            pl.BlockSpec((QT, NCOMBO), lambda i: (i, 0)),
            pl.BlockSpec((3 * NCOMBO, 256), lambda i: (0, 0)),
            pl.BlockSpec((1, 3 * NCOMBO), lambda i: (0, 0)),
            pl.BlockSpec((3, NCOMBO), lambda i: (0, 0)),
            pl.BlockSpec((NCOMBO, NCOMBO), lambda i: (0, 0)),
        ],
        out_specs=[
            pl.BlockSpec((QT, NTAP), lambda i: (i, 0)),
            pl.BlockSpec((QT, NTAP), lambda i: (i, 0)),
        ],
        out_shape=[
            jax.ShapeDtypeStruct((bq, NTAP), jnp.int32),
            jax.ShapeDtypeStruct((bq, NTAP), jnp.float32),
        ],
    )(qf, ref48, wcat, bcat, _lane_consts(), _group_mask())


VT = 384


def _vproj_body(v_ref, w_ref, b_ref, o_ref):
    f = v_ref[0].astype(jnp.bfloat16)
    o = lax.dot_general(f, w_ref[...], (((1,), (0,)), ((), ())),
                        preferred_element_type=jnp.float32)
    o_ref[0] = (o + b_ref[...]).astype(jnp.bfloat16)


def _vproj_call(vt, w_val_t, b_val):
    b = vt.shape[0]
    grid = (b, S_TOT // VT)
    return pl.pallas_call(
        _vproj_body,
        grid=grid,
        in_specs=[
            pl.BlockSpec((1, VT, 256), lambda i, j: (i, j, 0)),
            pl.BlockSpec((256, 256), lambda i, j: (0, 0)),
            pl.BlockSpec((1, 256), lambda i, j: (0, 0)),
        ],
        out_specs=pl.BlockSpec((1, VT, 256), lambda i, j: (i, j, 0)),
        out_shape=jax.ShapeDtypeStruct((b, S_TOT, 256), jnp.bfloat16),
    )(vt, w_val_t, b_val)


def _sc_body(table, idxh, wh, outh, idx_v, rows_v, w_v, out_v,
             sem0, sem1, semw0, semw1, semo0, semo1, semi0, semi1):
    wid = lax.axis_index("s") * 2 + lax.axis_index("c")
    n_chunks = idxh.shape[0] // NW
    base = wid * n_chunks
    sems = (sem0, sem1)
    semw = (semw0, semw1)
    semo = (semo0, semo1)
    semi = (semi0, semi1)

    def gathers(buf):
        for j in range(12):
            yield (table.at[idx_v.at[buf, j]],
                   rows_v.at[buf, pl.ds(j * 128, 128)])

    def icopy(ci, buf):
        return pltpu.make_async_copy(idxh.at[base + ci], idx_v.at[buf],
                                     semi[buf])

    def wcopy(ci, buf):
        return pltpu.make_async_copy(
            wh.at[pl.ds((base + ci) * CHUNK, CHUNK)], w_v.at[buf], semw[buf])

    def ocopy(ci, buf):
        return pltpu.make_async_copy(
            out_v.at[buf], outh.at[pl.ds((base + ci) * CHUNK, CHUNK)],
            semo[buf])

    def fire(ci, buf):
        icopy(ci, buf).wait()
        wcopy(ci, buf).start()
        for src, dst in gathers(buf):
            pltpu.async_copy(src, dst, sems[buf])

    def consume(ci, buf):
        for src, dst in gathers(buf):
            pltpu.make_async_copy(src, dst, sems[buf]).wait()

        @pl.when(ci + 2 < n_chunks)
        def _():
            icopy(ci + 2, buf).start()

        wcopy(ci, buf).wait()

        @pl.when(ci >= 2)
        def _():
            ocopy(ci - 2, buf).wait()

        def q_body(q, c2):
            ws = [w_v[buf, q, pl.ds(k * 16, 16)] for k in range(NTAP // 16)]
            for h in range(N_HEADS):
                acc0 = jnp.zeros((16,), jnp.float32)
                acc1 = jnp.zeros((16,), jnp.float32)
                for corner in range(4):
                    for j in range(6):
                        col = corner * NCOMBO + h * 6 + j
                        wv = jnp.full((16,), ws[col // 16][col % 16])
                        r = q * NTAP + col
                        wd = plsc.bitcast(rows_v[buf, r, :], jnp.int32)
                        f_ev = plsc.bitcast(wd << 16, jnp.float32)
                        f_od = plsc.bitcast(wd & jnp.int32(-65536), jnp.float32)
                        acc0 = acc0 + f_ev * wv
                        acc1 = acc1 + f_od * wv
                out_v[buf, q, pl.ds(h * 32, 16)] = acc0
                out_v[buf, q, pl.ds(h * 32 + 16, 16)] = acc1
            return c2

        lax.fori_loop(0, CHUNK, q_body, 0)
        ocopy(ci, buf).start()

    icopy(0, 0).start()
    icopy(1, 1).start()
    fire(0, 0)

    def it_body(it, carry):
        i0 = it * 2
        fire(i0 + 1, 1)
        consume(i0, 0)

        @pl.when(i0 + 2 < n_chunks)
        def _():
            fire(i0 + 2, 0)

        consume(i0 + 1, 1)
        return carry

    lax.fori_loop(0, n_chunks // 2, it_body, 0)
    ocopy(n_chunks - 2, 0).wait()
    ocopy(n_chunks - 1, 1).wait()


def _sc_call(table, idx3, w3, bq):
    mesh = plsc.VectorSubcoreMesh(core_axis_name="c", subcore_axis_name="s")
    f = pl.kernel(
        _sc_body,
        out_type=jax.ShapeDtypeStruct((bq, 256), jnp.float32),
        mesh=mesh,
        scratch_types=[
            pltpu.VMEM((2, 12, 128), jnp.int32),
            pltpu.VMEM((2, CHUNK * NTAP, HEAD_DIM), jnp.bfloat16),
            pltpu.VMEM((2, CHUNK, NTAP), jnp.float32),
            pltpu.VMEM((2, CHUNK, 256), jnp.float32),
            pltpu.SemaphoreType.DMA,
            pltpu.SemaphoreType.DMA,
            pltpu.SemaphoreType.DMA,
            pltpu.SemaphoreType.DMA,
            pltpu.SemaphoreType.DMA,
            pltpu.SemaphoreType.DMA,
            pltpu.SemaphoreType.DMA,
            pltpu.SemaphoreType.DMA,
        ],
        compiler_params=pltpu.CompilerParams(use_tc_tiling_on_sc=False,
                                             needs_layout_passes=False),
    )
    return f(table, idx3, w3)


def _oproj_body(a_ref, w_ref, b_ref, o_ref):
    o = lax.dot_general(a_ref[...], w_ref[...], (((1,), (1,)), ((), ())),
                        preferred_element_type=jnp.float32)
    o_ref[...] = o + b_ref[...]


def _oproj_call(acc, w_out, b_out):
    bq = acc.shape[0]
    return pl.pallas_call(
        _oproj_body,
        grid=(bq // QT,),
        in_specs=[
            pl.BlockSpec((QT, 256), lambda i: (i, 0)),
            pl.BlockSpec((256, 256), lambda i: (0, 0)),
            pl.BlockSpec((1, 256), lambda i: (0, 0)),
        ],
        out_specs=pl.BlockSpec((QT, 256), lambda i: (i, 0)),
        out_shape=jax.ShapeDtypeStruct((bq, 256), jnp.float32),
    )(acc, w_out, b_out)


def kernel(query, value_l0, value_l1, value_l2, reference_points,
           W_off, b_off, W_attn, b_attn, W_val, b_val, W_out, b_out):
    b, q, d = query.shape
    bq = b * q
    hq = bq // 2
    qf = query.reshape(bq, d)
    ref48 = jnp.broadcast_to(reference_points.reshape(bq, 1, 2),
                             (bq, NCOMBO // 2, 2)).reshape(bq, NCOMBO)
    wcat = jnp.concatenate([W_off[0::2], W_off[1::2], W_attn], axis=0)
    bcat = jnp.concatenate([b_off[0::2], b_off[1::2], b_attn],
                           axis=0).reshape(1, 3 * NCOMBO)

    vcat = jnp.concatenate([
        value_l0.reshape(b, d, LVL_W[0] * LVL_W[0]),
        value_l1.reshape(b, d, LVL_W[1] * LVL_W[1]),
        value_l2.reshape(b, d, LVL_W[2] * LVL_W[2]),
    ], axis=2).transpose(0, 2, 1)
    table = _vproj_call(vcat, W_val.T.astype(jnp.bfloat16), b_val.reshape(1, d))
    table = table.reshape(b * S_TOT * N_HEADS, HEAD_DIM)

    j = np.arange(HEAD_DIM)
    tr = 2 * (j % 16) + (j // 16)
    perm = (np.arange(N_HEADS)[:, None] * HEAD_DIM + tr[None, :]).reshape(-1)
    w_out_p = W_out[:, jnp.asarray(perm)]
    b_out2 = b_out.reshape(1, d)

    idx, wgt = _prep_call(qf, ref48, wcat, bcat, 0)
    acc = _sc_call(table, idx.reshape(bq // CHUNK, 12, 128), wgt, bq)
    out = _oproj_call(acc, w_out_p, b_out2)
    return out.reshape(b, q, d)
